# Initial kernel scaffold; baseline (speedup 1.0000x reference)
import jax
import jax.numpy as jnp
from jax.experimental import pallas as pl


def kernel(x, edge_index2, edge_attr2, batch, l1_W11, l1_W12, l1_W13, l1_W14, l1_Wc, l1_bc, l1_fc11_W, l1_fc11_b, l1_fc12_W, l1_fc12_b, l2_W11, l2_W12, l2_W13, l2_W14, l2_Wc, l2_bc, l2_fc11_W, l2_fc11_b, l2_fc12_W, l2_fc12_b, l3_W11, l3_W12, l3_W13, l3_W14, l3_Wc, l3_bc, l3_fc11_W, l3_fc11_b, l3_fc12_W, l3_fc12_b, fc1_W, fc1_b):
    raise NotImplementedError("write your pallas kernel here")


# speedup vs baseline: 12.4765x; 12.4765x over previous
import functools

import jax
import jax.numpy as jnp
from jax import lax
from jax.experimental import pallas as pl
from jax.experimental.pallas import tpu as pltpu
from jax.experimental.pallas import tpu_sc as plsc

_N = 10000
_E = 320000
_NE = 16
_NOUT1 = 32
_NOUT2 = 16
_NIN = _NOUT1 + _NOUT2
_NG = 200
_YW = _NE * _NOUT1

_NCORES = 2
_NSUB = 16
_NW = _NCORES * _NSUB
_EPW = _E // _NW
_C = 80
_NCHUNK = _EPW // _C
_NPAD = 10240
_RPT = _NPAD // _NSUB

_BE = 4000
_BN = 2000


@functools.partial(
    pl.kernel,
    out_type=jax.ShapeDtypeStruct((_NCORES, _NPAD, _NOUT1), jnp.float32),
    mesh=plsc.VectorSubcoreMesh(core_axis_name="c", subcore_axis_name="s"),
    compiler_params=pltpu.CompilerParams(use_tc_tiling_on_sc=False),
    scratch_types=[
        pltpu.VMEM((_C,), jnp.int32),
        pltpu.VMEM((_C,), jnp.int32),
        pltpu.VMEM((_C, _NE), jnp.float32),
        pltpu.VMEM((_C, _YW), jnp.float32),
        pltpu.VMEM((_C, _NOUT1), jnp.float32),
        pltpu.VMEM((_RPT, _NOUT1), jnp.float32),
        pltpu.VMEM_SHARED((_NPAD, _NOUT1), jnp.float32),
        pltpu.SemaphoreType.DMA,
    ],
)
def _sc_conv(y_hbm, ea_hbm, src_hbm, dst_hbm, out_hbm,
             srcv, dstv, eav, rows, msg, zbuf, acc, sem):
    cid = lax.axis_index("c")
    sid = lax.axis_index("s")
    wid = cid * _NSUB + sid
    zero16 = jnp.zeros((16,), jnp.float32)

    def zrow(r, carry):
        zbuf[r, pl.ds(0, 16)] = zero16
        zbuf[r, pl.ds(16, 16)] = zero16
        return carry
    lax.fori_loop(0, _RPT, zrow, 0)
    pltpu.sync_copy(zbuf, acc.at[pl.ds(sid * _RPT, _RPT)])
    plsc.subcore_barrier()

    e_base = wid * _EPW

    def chunk(i, carry):
        e0 = pl.multiple_of(e_base + i * _C, 8)
        pltpu.sync_copy(src_hbm.at[pl.ds(e0, _C)], srcv)
        pltpu.sync_copy(dst_hbm.at[pl.ds(e0, _C)], dstv)
        pltpu.sync_copy(ea_hbm.at[pl.ds(e0, _C)], eav)
        pltpu.async_copy(y_hbm.at[srcv], rows, sem).wait()

        def edge(e, ecarry):
            m0 = jnp.zeros((16,), jnp.float32)
            m1 = jnp.zeros((16,), jnp.float32)
            ev = eav[e, pl.ds(0, _NE)]
            for k in range(_NE):
                a = ev[k]
                m0 = m0 + a * rows[e, pl.ds(k * _NOUT1, 16)]
                m1 = m1 + a * rows[e, pl.ds(k * _NOUT1 + 16, 16)]
            msg[e, pl.ds(0, 16)] = m0
            msg[e, pl.ds(16, 16)] = m1
            return ecarry
        lax.fori_loop(0, _C, edge, 0)

        pltpu.sync_copy(msg, acc.at[dstv], add=True)
        return carry
    lax.fori_loop(0, _NCHUNK, chunk, 0)

    plsc.subcore_barrier()
    pltpu.sync_copy(acc.at[pl.ds(sid * _RPT, _RPT)],
                    out_hbm.at[cid, pl.ds(sid * _RPT, _RPT)])


def _edge_mlp(edge_attr, wfirst, w14_1, w14_2, w14_3):
    def body(e_ref, wf_ref, wa_ref, wb_ref, wc_ref, o1_ref, o2_ref, o3_ref):
        t = jnp.dot(e_ref[...], wf_ref[...], preferred_element_type=jnp.float32)
        for li, (w_ref, o_ref) in enumerate(
                [(wa_ref, o1_ref), (wb_ref, o2_ref), (wc_ref, o3_ref)]):
            s = t[:, li * 96:(li + 1) * 96]
            tmp = jnp.concatenate(
                [jnp.maximum(s[:, :32], 0.0),
                 jnp.maximum(s[:, 32:64], 0.0) * jnp.maximum(s[:, 64:96], 0.0)],
                axis=1)
            o_ref[...] = jnp.maximum(
                jnp.dot(tmp, w_ref[...], preferred_element_type=jnp.float32), 0.0)

    grid = _E // _BE
    espec = pl.BlockSpec((_BE, _NE), lambda i: (i, 0))
    wspec = lambda shp: pl.BlockSpec(shp, lambda i: (0, 0))
    ospec = pl.BlockSpec((_BE, _NE), lambda i: (i, 0))
    return pl.pallas_call(
        body,
        grid=(grid,),
        in_specs=[espec, wspec((_NE, 288)), wspec((64, _NE)),
                  wspec((64, _NE)), wspec((64, _NE))],
        out_specs=[ospec, ospec, ospec],
        out_shape=[jax.ShapeDtypeStruct((_E, _NE), jnp.float32)] * 3,
    )(edge_attr, wfirst, w14_1, w14_2, w14_3)


def _dense_stage(x, wc_flat, fcw, fcb):
    ninp = x.shape[1]

    def body(x_ref, wc_ref, fw_ref, fb_ref, y_ref, x2_ref):
        xb = x_ref[...]
        y_ref[...] = jnp.dot(xb, wc_ref[...], preferred_element_type=jnp.float32)
        t = jnp.dot(xb, fw_ref[...], preferred_element_type=jnp.float32) + fb_ref[...]
        x2_ref[...] = jnp.maximum(t[:, :_NOUT2], 0.0) * jnp.maximum(t[:, _NOUT2:], 0.0)

    grid = _N // _BN
    return pl.pallas_call(
        body,
        grid=(grid,),
        in_specs=[pl.BlockSpec((_BN, ninp), lambda i: (i, 0)),
                  pl.BlockSpec((ninp, _YW), lambda i: (0, 0)),
                  pl.BlockSpec((ninp, 2 * _NOUT2), lambda i: (0, 0)),
                  pl.BlockSpec((1, 2 * _NOUT2), lambda i: (0, 0))],
        out_specs=[pl.BlockSpec((_BN, _YW), lambda i: (i, 0)),
                   pl.BlockSpec((_BN, _NOUT2), lambda i: (i, 0))],
        out_shape=[jax.ShapeDtypeStruct((_N, _YW), jnp.float32),
                   jax.ShapeDtypeStruct((_N, _NOUT2), jnp.float32)],
    )(x, wc_flat, fcw, fcb)


def _assemble(partials, bc, x2):
    def body(p_ref, bc_ref, x2_ref, x_ref):
        s = p_ref[0] + p_ref[1] + bc_ref[...]
        x_ref[...] = jnp.concatenate(
            [jnp.maximum(s, 0.0), x2_ref[...]], axis=1)

    grid = _N // _BN
    return pl.pallas_call(
        body,
        grid=(grid,),
        in_specs=[pl.BlockSpec((_NCORES, _BN, _NOUT1), lambda i: (0, i, 0)),
                  pl.BlockSpec((1, _NOUT1), lambda i: (0, 0)),
                  pl.BlockSpec((_BN, _NOUT2), lambda i: (i, 0))],
        out_specs=pl.BlockSpec((_BN, _NIN), lambda i: (i, 0)),
        out_shape=jax.ShapeDtypeStruct((_N, _NIN), jnp.float32),
    )(partials, bc, x2)


def _pool_head(partials, bc, x2, batch3d, fc_w, fc_b):
    def body(p_ref, bc_ref, x2_ref, b_ref, fw_ref, fb_ref, o_ref, acc_ref):
        i = pl.program_id(0)

        @pl.when(i == 0)
        def _():
            acc_ref[...] = jnp.zeros_like(acc_ref)

        s = p_ref[0] + p_ref[1] + bc_ref[...]
        xb = jnp.concatenate([jnp.maximum(s, 0.0), x2_ref[...]], axis=1)
        onehot = (b_ref[0] == lax.broadcasted_iota(jnp.int32, (_NG, 1), 0)
                  ).astype(jnp.float32)
        acc_ref[...] += jnp.dot(onehot, xb, preferred_element_type=jnp.float32)

        @pl.when(i == pl.num_programs(0) - 1)
        def _():
            o_ref[...] = jnp.tanh(
                jnp.dot(acc_ref[...], fw_ref[...],
                        preferred_element_type=jnp.float32) + fb_ref[...])

    grid = _N // _BN
    return pl.pallas_call(
        body,
        grid=(grid,),
        in_specs=[pl.BlockSpec((_NCORES, _BN, _NOUT1), lambda i: (0, i, 0)),
                  pl.BlockSpec((1, _NOUT1), lambda i: (0, 0)),
                  pl.BlockSpec((_BN, _NOUT2), lambda i: (i, 0)),
                  pl.BlockSpec((1, 1, _BN), lambda i: (i, 0, 0)),
                  pl.BlockSpec((_NIN, 10), lambda i: (0, 0)),
                  pl.BlockSpec((1, 10), lambda i: (0, 0))],
        out_specs=pl.BlockSpec((_NG, 10), lambda i: (0, 0)),
        out_shape=jax.ShapeDtypeStruct((_NG, 10), jnp.float32),
        scratch_shapes=[pltpu.VMEM((_NG, _NIN), jnp.float32)],
    )(partials, bc, x2, batch3d, fc_w, fc_b)


def kernel(x, edge_index2, edge_attr2, batch,
           l1_W11, l1_W12, l1_W13, l1_W14, l1_Wc, l1_bc,
           l1_fc11_W, l1_fc11_b, l1_fc12_W, l1_fc12_b,
           l2_W11, l2_W12, l2_W13, l2_W14, l2_Wc, l2_bc,
           l2_fc11_W, l2_fc11_b, l2_fc12_W, l2_fc12_b,
           l3_W11, l3_W12, l3_W13, l3_W14, l3_Wc, l3_bc,
           l3_fc11_W, l3_fc11_b, l3_fc12_W, l3_fc12_b,
           fc1_W, fc1_b):
    src = edge_index2[0]
    dst = edge_index2[1]

    wfirst = jnp.concatenate(
        [l1_W11, l1_W12, l1_W13, l2_W11, l2_W12, l2_W13,
         l3_W11, l3_W12, l3_W13], axis=1)
    ea1, ea2, ea3 = _edge_mlp(edge_attr2, wfirst, l1_W14, l2_W14, l3_W14)

    layers = [
        (ea1, l1_Wc, l1_bc, l1_fc11_W, l1_fc11_b, l1_fc12_W, l1_fc12_b),
        (ea2, l2_Wc, l2_bc, l2_fc11_W, l2_fc11_b, l2_fc12_W, l2_fc12_b),
        (ea3, l3_Wc, l3_bc, l3_fc11_W, l3_fc11_b, l3_fc12_W, l3_fc12_b),
    ]

    xcur = x
    batch3d = batch.reshape(_N // _BN, 1, _BN)
    out = None
    for li, (ea, wc, bc, f11w, f11b, f12w, f12b) in enumerate(layers):
        wc_flat = jnp.transpose(wc, (1, 0, 2)).reshape(wc.shape[1], _YW)
        fcw = jnp.concatenate([f11w, f12w], axis=1)
        fcb = jnp.concatenate([f11b, f12b]).reshape(1, 2 * _NOUT2)
        y, x2 = _dense_stage(xcur, wc_flat, fcw, fcb)
        partials = _sc_conv(y, ea, src, dst)
        if li < 2:
            xcur = _assemble(partials, bc.reshape(1, _NOUT1), x2)
        else:
            out = _pool_head(partials, bc.reshape(1, _NOUT1), x2, batch3d,
                             fc1_W, fc1_b.reshape(1, 10))
    return out


# speedup vs baseline: 19.7483x; 1.5828x over previous
import functools

import jax
import jax.numpy as jnp
from jax import lax
from jax.experimental import pallas as pl
from jax.experimental.pallas import tpu as pltpu
from jax.experimental.pallas import tpu_sc as plsc

_N = 10000
_E = 320000
_NE = 16
_NOUT1 = 32
_NOUT2 = 16
_NIN = _NOUT1 + _NOUT2
_NG = 200
_YW = _NE * _NOUT1

_NCORES = 2
_NSUB = 16
_NW = _NCORES * _NSUB
_EPW = _E // _NW
_C = 80
_NCHUNK = _EPW // _C
_NPAD = 10240
_RPT = _NPAD // _NSUB

_BE = 4000
_BN = 2000


@functools.partial(
    pl.kernel,
    out_type=jax.ShapeDtypeStruct((_NCORES, _NPAD, _NOUT1), jnp.float32),
    mesh=plsc.VectorSubcoreMesh(core_axis_name="c", subcore_axis_name="s"),
    compiler_params=pltpu.CompilerParams(use_tc_tiling_on_sc=False),
    scratch_types=[
        pltpu.VMEM((_C,), jnp.int32),
        pltpu.VMEM((_C,), jnp.int32),
        pltpu.VMEM((_C, _NE), jnp.float32),
        pltpu.VMEM((_C, _YW), jnp.float32),
        pltpu.VMEM((_C,), jnp.int32),
        pltpu.VMEM((_C,), jnp.int32),
        pltpu.VMEM((_C, _NE), jnp.float32),
        pltpu.VMEM((_C, _YW), jnp.float32),
        pltpu.VMEM((_C, _NOUT1), jnp.float32),
        pltpu.VMEM((_RPT, _NOUT1), jnp.float32),
        pltpu.VMEM_SHARED((_NPAD, _NOUT1), jnp.float32),
        pltpu.SemaphoreType.DMA,
        pltpu.SemaphoreType.DMA,
        pltpu.SemaphoreType.DMA,
    ],
)
def _sc_conv(y_hbm, ea_hbm, src_hbm, dst_hbm, out_hbm,
             srcv0, dstv0, eav0, rows0, srcv1, dstv1, eav1, rows1,
             msg, zbuf, acc, gsem0, gsem1, isem):
    cid = lax.axis_index("c")
    sid = lax.axis_index("s")
    wid = cid * _NSUB + sid
    zero16 = jnp.zeros((16,), jnp.float32)
    bufs = ((srcv0, dstv0, eav0, rows0, gsem0),
            (srcv1, dstv1, eav1, rows1, gsem1))

    def zrow(r, carry):
        zbuf[r, pl.ds(0, 16)] = zero16
        zbuf[r, pl.ds(16, 16)] = zero16
        return carry
    lax.fori_loop(0, _RPT, zrow, 0)
    pltpu.sync_copy(zbuf, acc.at[pl.ds(sid * _RPT, _RPT)])
    plsc.subcore_barrier()

    e_base = wid * _EPW

    def eoff(ci):
        return pl.multiple_of(e_base + (ci % _NCHUNK) * _C, 8)

    def fire_idx(ci, buf):
        srcv, dstv, eav, _, _ = buf
        e0 = eoff(ci)
        pltpu.async_copy(src_hbm.at[pl.ds(e0, _C)], srcv, isem)
        pltpu.async_copy(dst_hbm.at[pl.ds(e0, _C)], dstv, isem)
        pltpu.async_copy(ea_hbm.at[pl.ds(e0, _C)], eav, isem)

    def wait_idx(ci, buf):
        srcv, dstv, eav, _, _ = buf
        e0 = eoff(ci)
        pltpu.make_async_copy(src_hbm.at[pl.ds(e0, _C)], srcv, isem).wait()
        pltpu.make_async_copy(dst_hbm.at[pl.ds(e0, _C)], dstv, isem).wait()
        pltpu.make_async_copy(ea_hbm.at[pl.ds(e0, _C)], eav, isem).wait()

    def fire_gather(buf):
        srcv, _, _, rows, gsem = buf
        pltpu.async_copy(y_hbm.at[srcv], rows, gsem)

    def wait_gather(buf):
        srcv, _, _, rows, gsem = buf
        pltpu.make_async_copy(y_hbm.at[srcv], rows, gsem).wait()

    def compute_scatter(buf):
        _, dstv, eav, rows, _ = buf

        def edge(e, ecarry):
            m0 = jnp.zeros((16,), jnp.float32)
            m1 = jnp.zeros((16,), jnp.float32)
            ev = eav[e, pl.ds(0, _NE)]
            for k in range(_NE):
                a = ev[k]
                m0 = m0 + a * rows[e, pl.ds(k * _NOUT1, 16)]
                m1 = m1 + a * rows[e, pl.ds(k * _NOUT1 + 16, 16)]
            msg[e, pl.ds(0, 16)] = m0
            msg[e, pl.ds(16, 16)] = m1
            return ecarry
        lax.fori_loop(0, _C, edge, 0)
        pltpu.sync_copy(msg, acc.at[dstv], add=True)

    def step(ci, cur, nxt):
        wait_idx(ci + 1, nxt)
        wait_gather(cur)
        fire_gather(nxt)
        compute_scatter(cur)
        fire_idx(ci + 2, cur)

    fire_idx(0, bufs[0])
    wait_idx(0, bufs[0])
    fire_gather(bufs[0])
    fire_idx(1, bufs[1])

    def pair(i, carry):
        step(2 * i, bufs[0], bufs[1])
        step(2 * i + 1, bufs[1], bufs[0])
        return carry
    lax.fori_loop(0, (_NCHUNK - 1) // 2, pair, 0)
    step(_NCHUNK - 1, bufs[0], bufs[1])

    wait_gather(bufs[1])
    wait_idx(0, bufs[0])

    plsc.subcore_barrier()
    pltpu.sync_copy(acc.at[pl.ds(sid * _RPT, _RPT)],
                    out_hbm.at[cid, pl.ds(sid * _RPT, _RPT)])


def _edge_mlp(edge_attr, wfirst, w14_1, w14_2, w14_3):
    def body(e_ref, wf_ref, wa_ref, wb_ref, wc_ref, o1_ref, o2_ref, o3_ref):
        t = jnp.dot(e_ref[...], wf_ref[...], preferred_element_type=jnp.float32)
        for li, (w_ref, o_ref) in enumerate(
                [(wa_ref, o1_ref), (wb_ref, o2_ref), (wc_ref, o3_ref)]):
            s = t[:, li * 96:(li + 1) * 96]
            tmp = jnp.concatenate(
                [jnp.maximum(s[:, :32], 0.0),
                 jnp.maximum(s[:, 32:64], 0.0) * jnp.maximum(s[:, 64:96], 0.0)],
                axis=1)
            o_ref[...] = jnp.maximum(
                jnp.dot(tmp, w_ref[...], preferred_element_type=jnp.float32), 0.0)

    grid = _E // _BE
    espec = pl.BlockSpec((_BE, _NE), lambda i: (i, 0))
    wspec = lambda shp: pl.BlockSpec(shp, lambda i: (0, 0))
    ospec = pl.BlockSpec((_BE, _NE), lambda i: (i, 0))
    return pl.pallas_call(
        body,
        grid=(grid,),
        in_specs=[espec, wspec((_NE, 288)), wspec((64, _NE)),
                  wspec((64, _NE)), wspec((64, _NE))],
        out_specs=[ospec, ospec, ospec],
        out_shape=[jax.ShapeDtypeStruct((_E, _NE), jnp.float32)] * 3,
    )(edge_attr, wfirst, w14_1, w14_2, w14_3)


def _dense_stage(x, wc_flat, fcw, fcb):
    ninp = x.shape[1]

    def body(x_ref, wc_ref, fw_ref, fb_ref, y_ref, x2_ref):
        xb = x_ref[...]
        y_ref[...] = jnp.dot(xb, wc_ref[...], preferred_element_type=jnp.float32)
        t = jnp.dot(xb, fw_ref[...], preferred_element_type=jnp.float32) + fb_ref[...]
        x2_ref[...] = jnp.maximum(t[:, :_NOUT2], 0.0) * jnp.maximum(t[:, _NOUT2:], 0.0)

    grid = _N // _BN
    return pl.pallas_call(
        body,
        grid=(grid,),
        in_specs=[pl.BlockSpec((_BN, ninp), lambda i: (i, 0)),
                  pl.BlockSpec((ninp, _YW), lambda i: (0, 0)),
                  pl.BlockSpec((ninp, 2 * _NOUT2), lambda i: (0, 0)),
                  pl.BlockSpec((1, 2 * _NOUT2), lambda i: (0, 0))],
        out_specs=[pl.BlockSpec((_BN, _YW), lambda i: (i, 0)),
                   pl.BlockSpec((_BN, _NOUT2), lambda i: (i, 0))],
        out_shape=[jax.ShapeDtypeStruct((_N, _YW), jnp.float32),
                   jax.ShapeDtypeStruct((_N, _NOUT2), jnp.float32)],
    )(x, wc_flat, fcw, fcb)


def _assemble(partials, bc, x2):
    def body(p_ref, bc_ref, x2_ref, x_ref):
        s = p_ref[0] + p_ref[1] + bc_ref[...]
        x_ref[...] = jnp.concatenate(
            [jnp.maximum(s, 0.0), x2_ref[...]], axis=1)

    grid = _N // _BN
    return pl.pallas_call(
        body,
        grid=(grid,),
        in_specs=[pl.BlockSpec((_NCORES, _BN, _NOUT1), lambda i: (0, i, 0)),
                  pl.BlockSpec((1, _NOUT1), lambda i: (0, 0)),
                  pl.BlockSpec((_BN, _NOUT2), lambda i: (i, 0))],
        out_specs=pl.BlockSpec((_BN, _NIN), lambda i: (i, 0)),
        out_shape=jax.ShapeDtypeStruct((_N, _NIN), jnp.float32),
    )(partials, bc, x2)


def _pool_head(partials, bc, x2, batch3d, fc_w, fc_b):
    def body(p_ref, bc_ref, x2_ref, b_ref, fw_ref, fb_ref, o_ref, acc_ref):
        i = pl.program_id(0)

        @pl.when(i == 0)
        def _():
            acc_ref[...] = jnp.zeros_like(acc_ref)

        s = p_ref[0] + p_ref[1] + bc_ref[...]
        xb = jnp.concatenate([jnp.maximum(s, 0.0), x2_ref[...]], axis=1)
        onehot = (b_ref[0] == lax.broadcasted_iota(jnp.int32, (_NG, 1), 0)
                  ).astype(jnp.float32)
        acc_ref[...] += jnp.dot(onehot, xb, preferred_element_type=jnp.float32)

        @pl.when(i == pl.num_programs(0) - 1)
        def _():
            o_ref[...] = jnp.tanh(
                jnp.dot(acc_ref[...], fw_ref[...],
                        preferred_element_type=jnp.float32) + fb_ref[...])

    grid = _N // _BN
    return pl.pallas_call(
        body,
        grid=(grid,),
        in_specs=[pl.BlockSpec((_NCORES, _BN, _NOUT1), lambda i: (0, i, 0)),
                  pl.BlockSpec((1, _NOUT1), lambda i: (0, 0)),
                  pl.BlockSpec((_BN, _NOUT2), lambda i: (i, 0)),
                  pl.BlockSpec((1, 1, _BN), lambda i: (i, 0, 0)),
                  pl.BlockSpec((_NIN, 10), lambda i: (0, 0)),
                  pl.BlockSpec((1, 10), lambda i: (0, 0))],
        out_specs=pl.BlockSpec((_NG, 10), lambda i: (0, 0)),
        out_shape=jax.ShapeDtypeStruct((_NG, 10), jnp.float32),
        scratch_shapes=[pltpu.VMEM((_NG, _NIN), jnp.float32)],
    )(partials, bc, x2, batch3d, fc_w, fc_b)


def kernel(x, edge_index2, edge_attr2, batch,
           l1_W11, l1_W12, l1_W13, l1_W14, l1_Wc, l1_bc,
           l1_fc11_W, l1_fc11_b, l1_fc12_W, l1_fc12_b,
           l2_W11, l2_W12, l2_W13, l2_W14, l2_Wc, l2_bc,
           l2_fc11_W, l2_fc11_b, l2_fc12_W, l2_fc12_b,
           l3_W11, l3_W12, l3_W13, l3_W14, l3_Wc, l3_bc,
           l3_fc11_W, l3_fc11_b, l3_fc12_W, l3_fc12_b,
           fc1_W, fc1_b):
    src = edge_index2[0]
    dst = edge_index2[1]

    wfirst = jnp.concatenate(
        [l1_W11, l1_W12, l1_W13, l2_W11, l2_W12, l2_W13,
         l3_W11, l3_W12, l3_W13], axis=1)
    ea1, ea2, ea3 = _edge_mlp(edge_attr2, wfirst, l1_W14, l2_W14, l3_W14)

    layers = [
        (ea1, l1_Wc, l1_bc, l1_fc11_W, l1_fc11_b, l1_fc12_W, l1_fc12_b),
        (ea2, l2_Wc, l2_bc, l2_fc11_W, l2_fc11_b, l2_fc12_W, l2_fc12_b),
        (ea3, l3_Wc, l3_bc, l3_fc11_W, l3_fc11_b, l3_fc12_W, l3_fc12_b),
    ]

    xcur = x
    batch3d = batch.reshape(_N // _BN, 1, _BN)
    out = None
    for li, (ea, wc, bc, f11w, f11b, f12w, f12b) in enumerate(layers):
        wc_flat = jnp.transpose(wc, (1, 0, 2)).reshape(wc.shape[1], _YW)
        fcw = jnp.concatenate([f11w, f12w], axis=1)
        fcb = jnp.concatenate([f11b, f12b]).reshape(1, 2 * _NOUT2)
        y, x2 = _dense_stage(xcur, wc_flat, fcw, fcb)
        partials = _sc_conv(y, ea, src, dst)
        if li < 2:
            xcur = _assemble(partials, bc.reshape(1, _NOUT1), x2)
        else:
            out = _pool_head(partials, bc.reshape(1, _NOUT1), x2, batch3d,
                             fc1_W, fc1_b.reshape(1, 10))
    return out
